# baseline (speedup 1.0000x reference)
import functools

import jax
import jax.numpy as jnp
from jax import lax
from jax.experimental import pallas as pl
from jax.experimental.pallas import tpu as pltpu
from jax.experimental.pallas import tpu_sc as plsc

_NBUF = 8
_SPAD = 56


def _make_sc_gather(V, D, B):
    info = plsc.get_sparse_core_info()
    nw = info.num_cores * info.num_subcores
    spw = B // nw
    n_outer = spw // _NBUF
    group = _NBUF * _SPAD
    assert spw % _NBUF == 0
    mesh = plsc.VectorSubcoreMesh(core_axis_name="c", subcore_axis_name="s")

    @functools.partial(
        pl.kernel,
        mesh=mesh,
        out_type=jax.ShapeDtypeStruct((B * _SPAD, D), jnp.float32),
        scratch_types=[
            pltpu.VMEM((spw, _SPAD), jnp.int32),
            pltpu.VMEM((group, D), jnp.float32),
        ] + [pltpu.SemaphoreType.DMA] * _NBUF,
        compiler_params=pltpu.CompilerParams(use_tc_tiling_on_sc=False),
    )
    def gather(idx_hbm, table_hbm, out_hbm, idx_v, rows_v, *sems):
        wid = lax.axis_index("s") * info.num_cores + lax.axis_index("c")
        sample_base = wid * spw
        pltpu.sync_copy(
            idx_hbm.at[pl.ds(sample_base, spw), pl.ds(0, _SPAD)], idx_v)

        def body(g, carry):
            s0 = g * _NBUF
            cps = []
            for j in range(_NBUF):
                cps.append(pltpu.async_copy(
                    table_hbm.at[idx_v.at[s0 + j]],
                    rows_v.at[pl.ds(j * _SPAD, _SPAD)],
                    sems[j]))
            for cp in cps:
                cp.wait()
            pltpu.sync_copy(
                rows_v,
                out_hbm.at[pl.ds((sample_base + s0) * _SPAD, group)])
            return carry

        lax.fori_loop(0, n_outer, body, 0)

    return gather


def _make_tc_head(Bb, K, C, BB):
    def body(e_ref, w_ref, b_ref, o_ref):
        e = e_ref[...].astype(jnp.bfloat16)
        logits = lax.dot_general(e, w_ref[...], (((1,), (1,)), ((), ())),
                                 preferred_element_type=jnp.float32)
        logits = logits + b_ref[...]
        m = jnp.max(logits, axis=-1, keepdims=True)
        p = jnp.exp(logits - m)
        o_ref[...] = p / jnp.sum(p, axis=-1, keepdims=True)

    return pl.pallas_call(
        body,
        grid=(Bb // BB,),
        in_specs=[
            pl.BlockSpec((BB, K), lambda i: (i, 0)),
            pl.BlockSpec((C, K), lambda i: (0, 0)),
            pl.BlockSpec((1, C), lambda i: (0, 0)),
        ],
        out_specs=pl.BlockSpec((BB, C), lambda i: (i, 0)),
        out_shape=jax.ShapeDtypeStruct((Bb, C), jnp.float32),
    )


def kernel(x, embed, W, b):
    B, S = x.shape
    V, D = embed.shape
    C = W.shape[0]
    k_pad = _SPAD * D

    x128 = jnp.pad(x.astype(jnp.int32), ((0, 0), (0, 128 - S)))

    w_pad = jnp.pad(W, ((0, 0), (0, k_pad - S * D))).astype(jnp.bfloat16)

    e = _make_sc_gather(V, D, B)(x128, embed)
    e2 = e.reshape(B, k_pad)
    head = _make_tc_head(B, k_pad, C, 512)
    return head(e2, w_pad, b.reshape(1, C))


# speedup vs baseline: 2.3016x; 2.3016x over previous
import functools

import jax
import jax.numpy as jnp
from jax import lax
from jax.experimental import pallas as pl
from jax.experimental.pallas import tpu as pltpu
from jax.experimental.pallas import tpu_sc as plsc

_NBUF = 8
_GROWS = 128


def _make_sc_gather(V, D, N):
    info = plsc.get_sparse_core_info()
    nw = info.num_cores * info.num_subcores
    rpw = N // nw
    group = _NBUF * _GROWS
    n_outer = rpw // group
    assert rpw % group == 0
    mesh = plsc.VectorSubcoreMesh(core_axis_name="c", subcore_axis_name="s")

    @functools.partial(
        pl.kernel,
        mesh=mesh,
        out_type=jax.ShapeDtypeStruct((N, D), jnp.float32),
        scratch_types=[
            pltpu.VMEM((rpw,), jnp.int32),
            pltpu.VMEM((group, D), jnp.float32),
        ] + [pltpu.SemaphoreType.DMA] * _NBUF,
        compiler_params=pltpu.CompilerParams(use_tc_tiling_on_sc=False),
    )
    def gather(idx_hbm, table_hbm, out_hbm, idx_v, rows_v, *sems):
        wid = lax.axis_index("s") * info.num_cores + lax.axis_index("c")
        row_base = wid * rpw
        pltpu.sync_copy(idx_hbm.at[pl.ds(row_base, rpw)], idx_v)

        def body(g, carry):
            r0 = g * group
            cps = []
            for j in range(_NBUF):
                cps.append(pltpu.async_copy(
                    table_hbm.at[idx_v.at[pl.ds(r0 + j * _GROWS, _GROWS)]],
                    rows_v.at[pl.ds(j * _GROWS, _GROWS)],
                    sems[j]))
            for cp in cps:
                cp.wait()
            pltpu.sync_copy(
                rows_v, out_hbm.at[pl.ds(row_base + r0, group)])
            return carry

        lax.fori_loop(0, n_outer, body, 0)

    return gather


def _make_tc_head(Bb, K, C, BB):
    def body(e_ref, w_ref, b_ref, o_ref):
        e = e_ref[...].astype(jnp.bfloat16)
        logits = lax.dot_general(e, w_ref[...], (((1,), (1,)), ((), ())),
                                 preferred_element_type=jnp.float32)
        logits = logits + b_ref[...]
        m = jnp.max(logits, axis=-1, keepdims=True)
        p = jnp.exp(logits - m)
        o_ref[...] = p / jnp.sum(p, axis=-1, keepdims=True)

    return pl.pallas_call(
        body,
        grid=(Bb // BB,),
        in_specs=[
            pl.BlockSpec((BB, K), lambda i: (i, 0)),
            pl.BlockSpec((C, K), lambda i: (0, 0)),
            pl.BlockSpec((1, C), lambda i: (0, 0)),
        ],
        out_specs=pl.BlockSpec((BB, C), lambda i: (i, 0)),
        out_shape=jax.ShapeDtypeStruct((Bb, C), jnp.float32),
    )


def kernel(x, embed, W, b):
    B, S = x.shape
    V, D = embed.shape
    C = W.shape[0]
    N = B * S

    idx = x.astype(jnp.int32).reshape(N)
    e = _make_sc_gather(V, D, N)(idx, embed)
    e2 = e.reshape(B, S * D)
    head = _make_tc_head(B, S * D, C, 512)
    return head(e2, W.astype(jnp.bfloat16), b.reshape(1, C))


# speedup vs baseline: 2.3298x; 1.0123x over previous
import functools

import jax
import jax.numpy as jnp
from jax import lax
from jax.experimental import pallas as pl
from jax.experimental.pallas import tpu as pltpu
from jax.experimental.pallas import tpu_sc as plsc

_NBUF = 8
_GROWS = 128


def _make_sc_gather(V, D, N):
    info = plsc.get_sparse_core_info()
    nw = info.num_cores * info.num_subcores
    rpw = N // nw
    group = _NBUF * _GROWS
    n_outer = rpw // group
    assert rpw % group == 0
    mesh = plsc.VectorSubcoreMesh(core_axis_name="c", subcore_axis_name="s")

    @functools.partial(
        pl.kernel,
        mesh=mesh,
        out_type=jax.ShapeDtypeStruct((N, D), jnp.float32),
        scratch_types=[
            pltpu.VMEM((rpw,), jnp.int32),
            pltpu.VMEM((2 * group, D), jnp.float32),
        ] + [pltpu.SemaphoreType.DMA] * (2 * _NBUF + 2),
        compiler_params=pltpu.CompilerParams(use_tc_tiling_on_sc=False),
    )
    def gather(idx_hbm, table_hbm, out_hbm, idx_v, rows_v, *sems):
        wid = lax.axis_index("s") * info.num_cores + lax.axis_index("c")
        row_base = wid * rpw
        pltpu.sync_copy(idx_hbm.at[pl.ds(row_base, rpw)], idx_v)

        wb_sems = sems[2 * _NBUF:]
        wb_cps = [None, None]
        for g in range(n_outer):
            ping = g % 2
            r0 = g * group
            if wb_cps[ping] is not None:
                wb_cps[ping].wait()
            cps = []
            for j in range(_NBUF):
                cps.append(pltpu.async_copy(
                    table_hbm.at[idx_v.at[pl.ds(r0 + j * _GROWS, _GROWS)]],
                    rows_v.at[pl.ds(ping * group + j * _GROWS, _GROWS)],
                    sems[ping * _NBUF + j]))
            for cp in cps:
                cp.wait()
            wb_cps[ping] = pltpu.async_copy(
                rows_v.at[pl.ds(ping * group, group)],
                out_hbm.at[pl.ds(row_base + r0, group)],
                wb_sems[ping])
        for cp in wb_cps:
            if cp is not None:
                cp.wait()

    return gather


def _make_tc_head(Bb, K, C, BB):
    def body(e_ref, w_ref, b_ref, o_ref):
        e = e_ref[...].astype(jnp.bfloat16)
        logits = lax.dot_general(e, w_ref[...], (((1,), (1,)), ((), ())),
                                 preferred_element_type=jnp.float32)
        logits = logits + b_ref[...]
        m = jnp.max(logits, axis=-1, keepdims=True)
        p = jnp.exp(logits - m)
        o_ref[...] = p / jnp.sum(p, axis=-1, keepdims=True)

    return pl.pallas_call(
        body,
        grid=(Bb // BB,),
        in_specs=[
            pl.BlockSpec((BB, K), lambda i: (i, 0)),
            pl.BlockSpec((C, K), lambda i: (0, 0)),
            pl.BlockSpec((1, C), lambda i: (0, 0)),
        ],
        out_specs=pl.BlockSpec((BB, C), lambda i: (i, 0)),
        out_shape=jax.ShapeDtypeStruct((Bb, C), jnp.float32),
    )


def kernel(x, embed, W, b):
    B, S = x.shape
    V, D = embed.shape
    C = W.shape[0]
    N = B * S

    idx = x.astype(jnp.int32).reshape(N)
    e = _make_sc_gather(V, D, N)(idx, embed)
    e2 = e.reshape(B, S * D)
    head = _make_tc_head(B, S * D, C, 512)
    return head(e2, W.astype(jnp.bfloat16), b.reshape(1, C))
